# baseline (speedup 1.0000x reference)
import jax
import jax.numpy as jnp
import numpy as np
from jax.experimental import pallas as pl
from jax.experimental.pallas import tpu as pltpu

B = 4
S = 4096
HIDDEN = 2048
H4 = HIDDEN // 4
K = S // 2
TILE = 1024
NTILES = (B * S) // TILE
RPT = S // TILE

_GROUP = (np.arange(B)[:, None] == (np.arange(NTILES)[None, :] // RPT)
          ).astype(np.float32)


def _rowsum(x, group, groupT):
    c = jnp.sum(x, axis=1, keepdims=True).astype(jnp.float32)
    c4 = jax.lax.dot_general(group, c, (((1,), (0,)), ((), ())),
                             precision=jax.lax.Precision.HIGHEST)
    return jax.lax.dot_general(groupT, c4, (((1,), (0,)), ((), ())),
                               precision=jax.lax.Precision.HIGHEST)


def _body(x_ref, w1_ref, b1_ref, w2t_ref, b2_ref, group_ref, w_ref, m_ref,
          scores_ref):
    i = pl.program_id(0)
    x = x_ref[...]
    h = jnp.dot(x, w1_ref[...], preferred_element_type=jnp.float32) + b1_ref[...]
    g = h * (0.5 * (jax.lax.erf(h * jnp.float32(0.7071067811865476)) + 1.0))
    logits = jax.lax.dot_general(
        w2t_ref[...], g, (((1,), (1,)), ((), ())),
        preferred_element_type=jnp.float32) + b2_ref[...]
    scores_ref[pl.ds(i, 1), :] = jax.nn.sigmoid(logits)

    @pl.when(i == NTILES - 1)
    def _select():
        group = group_ref[...]
        groupT = group.T
        s = scores_ref[...]
        key = jax.lax.bitcast_convert_type(s, jnp.int32)

        def step(it, p):
            b = 30 - it
            q = p | (1 << b)
            cnt = _rowsum(((key >> b) >= (q >> b)).astype(jnp.int32),
                          group, groupT)
            return jnp.where(cnt >= K, q, p)

        p = jax.lax.fori_loop(0, 31, step,
                              jnp.zeros((NTILES, 1), jnp.int32))

        gt = key > p
        eq = key == p
        need = (K - _rowsum(gt.astype(jnp.int32), group, groupT)
                ).astype(jnp.int32)

        col = (jax.lax.broadcasted_iota(jnp.int32, (NTILES, TILE), 0) % RPT
               ) * TILE + jax.lax.broadcasted_iota(jnp.int32, (NTILES, TILE), 1)
        lo = (S - 1) - col

        def step2(it, plo):
            b = 11 - it
            q = plo | (1 << b)
            cnt = _rowsum((eq & ((lo >> b) >= (q >> b))).astype(jnp.int32),
                          group, groupT)
            return jnp.where(cnt >= need, q, plo)

        plo = jax.lax.fori_loop(0, 12, step2,
                                jnp.zeros((NTILES, 1), jnp.int32))

        mask = gt | (eq & (lo >= plo))
        m_ref[...] = mask
        w_ref[...] = s * mask.astype(s.dtype)


@jax.jit
def kernel(hidden_states, W1, b1, W2, b2):
    x = hidden_states.reshape(B * S, HIDDEN)
    weights, mask = pl.pallas_call(
        _body,
        grid=(NTILES,),
        in_specs=[
            pl.BlockSpec((TILE, HIDDEN), lambda i: (i, 0)),
            pl.BlockSpec((HIDDEN, H4), lambda i: (0, 0)),
            pl.BlockSpec((1, H4), lambda i: (0, 0)),
            pl.BlockSpec((1, H4), lambda i: (0, 0)),
            pl.BlockSpec((1, 1), lambda i: (0, 0)),
            pl.BlockSpec((B, NTILES), lambda i: (0, 0)),
        ],
        out_specs=(
            pl.BlockSpec((NTILES, TILE), lambda i: (0, 0)),
            pl.BlockSpec((NTILES, TILE), lambda i: (0, 0)),
        ),
        out_shape=(
            jax.ShapeDtypeStruct((NTILES, TILE), jnp.float32),
            jax.ShapeDtypeStruct((NTILES, TILE), jnp.bool_),
        ),
        scratch_shapes=[pltpu.VMEM((NTILES, TILE), jnp.float32)],
        compiler_params=pltpu.CompilerParams(
            dimension_semantics=("arbitrary",)),
    )(x, W1, b1.reshape(1, H4), W2.reshape(1, H4), b2.reshape(1, 1),
      jnp.asarray(_GROUP))
    return weights.reshape(B, S), mask.reshape(B, S)


# speedup vs baseline: 1.2354x; 1.2354x over previous
import jax
import jax.numpy as jnp
from jax.experimental import pallas as pl
from jax.experimental.pallas import tpu as pltpu

B = 4
S = 4096
HIDDEN = 2048
H4 = HIDDEN // 4
K = S // 2
TILE = 1024
NTILES = (B * S) // TILE
RPT = S // TILE


def _body(x_ref, w1_ref, b1_ref, w2t_ref, b2_ref, w_ref, m_ref, scores_ref):
    i = pl.program_id(0)
    x = x_ref[...]
    h = jnp.dot(x, w1_ref[...], preferred_element_type=jnp.float32) + b1_ref[...]
    g = h * (0.5 * (jax.lax.erf(h * jnp.float32(0.7071067811865476)) + 1.0))
    logits = jax.lax.dot_general(
        w2t_ref[...], g, (((1,), (1,)), ((), ())),
        preferred_element_type=jnp.float32) + b2_ref[...]
    row = i // RPT
    col = pl.multiple_of((i % RPT) * TILE, TILE)
    scores_ref[pl.ds(row, 1), pl.ds(col, TILE)] = jax.nn.sigmoid(logits)

    @pl.when(i == NTILES - 1)
    def _select():
        s = scores_ref[...]
        key = jax.lax.bitcast_convert_type(s, jnp.int32)

        def step(it, p):
            b = 30 - it
            q = p | (1 << b)
            c = jnp.sum(((key >> b) >= (q >> b)).astype(jnp.int32), axis=1,
                        keepdims=True)
            return jnp.where(c >= K, q, p)

        p = jax.lax.fori_loop(0, 31, step, jnp.zeros((B, 1), jnp.int32))

        gt = key > p
        eq = key == p
        need = K - jnp.sum(gt.astype(jnp.int32), axis=1, keepdims=True)

        lo = (S - 1) - jax.lax.broadcasted_iota(jnp.int32, (B, S), 1)

        def step2(it, plo):
            b = 11 - it
            q = plo | (1 << b)
            c = jnp.sum((eq & ((lo >> b) >= (q >> b))).astype(jnp.int32),
                        axis=1, keepdims=True)
            return jnp.where(c >= need, q, plo)

        plo = jax.lax.fori_loop(0, 12, step2, jnp.zeros((B, 1), jnp.int32))

        mask = gt | (eq & (lo >= plo))
        m_ref[...] = mask
        w_ref[...] = s * mask.astype(s.dtype)


@jax.jit
def kernel(hidden_states, W1, b1, W2, b2):
    x = hidden_states.reshape(B * S, HIDDEN)
    weights, mask = pl.pallas_call(
        _body,
        grid=(NTILES,),
        in_specs=[
            pl.BlockSpec((TILE, HIDDEN), lambda i: (i, 0)),
            pl.BlockSpec((HIDDEN, H4), lambda i: (0, 0)),
            pl.BlockSpec((1, H4), lambda i: (0, 0)),
            pl.BlockSpec((1, H4), lambda i: (0, 0)),
            pl.BlockSpec((1, 1), lambda i: (0, 0)),
        ],
        out_specs=(
            pl.BlockSpec((B, S), lambda i: (0, 0)),
            pl.BlockSpec((B, S), lambda i: (0, 0)),
        ),
        out_shape=(
            jax.ShapeDtypeStruct((B, S), jnp.float32),
            jax.ShapeDtypeStruct((B, S), jnp.bool_),
        ),
        scratch_shapes=[pltpu.VMEM((B, S), jnp.float32)],
        compiler_params=pltpu.CompilerParams(
            dimension_semantics=("arbitrary",)),
    )(x, W1, b1.reshape(1, H4), W2.reshape(1, H4), b2.reshape(1, 1))
    return weights, mask
